# baseline (speedup 1.0000x reference)
import functools

import jax
import jax.numpy as jnp
from jax import lax
from jax.experimental import pallas as pl
from jax.experimental.pallas import tpu as pltpu
from jax.experimental.pallas import tpu_sc as plsc

H = 128
N_NODES = {"show": 10000, "performance": 100000, "song": 10000}
EDGE_TYPES = [
    ("show", "has_performance", "performance"),
    ("performance", "of_song", "song"),
    ("song", "transitioned_to", "song"),
    ("show", "setlist_neighbor", "show"),
    ("performance", "rev_has_performance", "show"),
    ("song", "rev_of_song", "performance"),
    ("song", "rev_transitioned_to", "song"),
]
N_EDGES = 100000

DST_GROUPS = {
    "show": ["setlist_neighbor", "rev_has_performance"],
    "performance": ["has_performance", "rev_of_song"],
    "song": ["of_song", "transitioned_to", "rev_transitioned_to"],
}

NC, NS = 2, 16
NW = NC * NS
E_PAD = 102400
EB = E_PAD // 128
SLAB = EB // NW
KB = 128
CHUNK = 5000
SP_ROWS = 5120
ZTR = 320


def _agg_body(with_counts, *refs):
    i = 0
    x_refs = {"show": refs[0], "performance": refs[1], "song": refs[2]}
    i = 3
    ei_refs = {}
    for (_, r, _) in EDGE_TYPES:
        ei_refs[r] = (refs[i], refs[i + 1])
        i += 2
    zb_hbm, ones_tab = refs[i], refs[i + 1]
    i += 2
    s_out = {}
    for (_, r, _) in EDGE_TYPES:
        s_out[r] = refs[i]
        i += 1
    c_out = {}
    if with_counts:
        for (_, r, _) in EDGE_TYPES:
            c_out[r] = refs[i]
            i += 1
    (S_sp, src_ids, dst_ids, csrc, cdst, rows, rows2, rows3,
     semA, semB, semC) = refs[i:]

    c = lax.axis_index("c")
    s = lax.axis_index("s")

    def zero_spmem():
        pltpu.sync_copy(zb_hbm, rows)
        r0 = s * ZTR
        for j in range(2):
            pltpu.sync_copy(rows, S_sp.at[pl.ds(r0 + j * KB, KB)])
        pltpu.sync_copy(rows.at[pl.ds(0, 64)], S_sp.at[pl.ds(r0 + 256, 64)])


    def wb(dst_s):
        @pl.when(s < NS - 1)
        def _wb_main():
            r0 = s * 312
            pltpu.sync_copy(S_sp.at[pl.ds(r0, 312)], dst_s(r0, 312))

        @pl.when(s == NS - 1)
        def _wb_tail():
            pltpu.sync_copy(S_sp.at[pl.ds(4680, 320)], dst_s(4680, 320))

    def agg_type(r, x_hbm, n_dst, out_ref, counts_mode):
        src_ref, dst_ref = ei_refs[r]
        iot = lax.iota(jnp.int32, 16)
        ncs = n_dst // (NC * CHUNK)

        def chunk_body(ch, _carry):
            lo = (c * ncs + ch) * CHUNK
            hi = lo + CHUNK
            zero_spmem()
            plsc.subcore_barrier()
            k = 0
            for hh in range(2):
                pltpu.sync_copy(src_ref.at[2 * s + hh], src_ids)
                pltpu.sync_copy(dst_ref.at[2 * s + hh], dst_ids)

                def comp(i2, kk):
                    rr = i2 // 8
                    cc = (i2 % 8) * 16
                    vd = dst_ids[rr, pl.ds(cc, 16)]
                    vs = src_ids[rr, pl.ds(cc, 16)]
                    m = (vd >= lo) & (vd < hi)
                    mi = m.astype(jnp.int32)
                    pos = kk + plsc.cumsum(mi) - mi
                    if not counts_mode:
                        plsc.store_scatter(csrc, [pos // KB, pos % KB], vs,
                                           mask=m)
                    plsc.store_scatter(cdst, [pos // KB, pos % KB], vd - lo,
                                       mask=m)
                    return kk + jnp.sum(mi)

                k = lax.fori_loop(0, SLAB * 8, comp, k)
            for t in range(8):
                pos = k + 16 * t + iot
                if not counts_mode:
                    plsc.store_scatter(csrc, [pos // KB, pos % KB],
                                       jnp.zeros((16,), jnp.int32))
                plsc.store_scatter(cdst, [pos // KB, pos % KB],
                                   jnp.full((16,), CHUNK, jnp.int32))
            nblk = (k + KB - 1) // KB

            if counts_mode:
                pltpu.sync_copy(ones_tab, rows)

                def blk(b, carry):
                    pltpu.sync_copy(rows, S_sp.at[cdst.at[b]], add=True)
                    return carry

                lax.fori_loop(0, nblk, blk, 0)
            else:
                bufs = ((rows, semA), (rows2, semB), (rows3, semC))
                pltpu.async_copy(x_hbm.at[csrc.at[0]], rows, semA)
                for j in (1, 2):
                    @pl.when(j < nblk)
                    def _gp(j=j):
                        pltpu.async_copy(x_hbm.at[csrc.at[j]], bufs[j][0],
                                         bufs[j][1])

                def trip(p, carry):
                    b0 = 3 * p
                    for j in range(3):
                        bj, (buf, sj) = b0 + j, bufs[j]

                        @pl.when(bj < nblk)
                        def _do(bj=bj, buf=buf, sj=sj):
                            pltpu.make_async_copy(
                                x_hbm.at[csrc.at[bj]], buf, sj).wait()
                            pltpu.sync_copy(buf, S_sp.at[cdst.at[bj]], add=True)

                            @pl.when(bj + 3 < nblk)
                            def _gn():
                                pltpu.async_copy(
                                    x_hbm.at[csrc.at[bj + 3]], buf, sj)

                    return carry

                lax.fori_loop(0, (nblk + 2) // 3, trip, 0)
            plsc.subcore_barrier()
            wb(lambda r0, n: out_ref.at[pl.ds(lo + r0, n)])
            plsc.subcore_barrier()
            return _carry

        lax.fori_loop(0, ncs, chunk_body, 0)

    for (st, r, d) in EDGE_TYPES:
        agg_type(r, x_refs[st], N_NODES[d], s_out[r], False)
        if with_counts:
            agg_type(r, ones_tab, N_NODES[d], c_out[r], True)


def _make_agg(with_counts):
    mesh = plsc.VectorSubcoreMesh(
        core_axis_name="c", subcore_axis_name="s", num_cores=NC, num_subcores=NS)
    out_type = [jax.ShapeDtypeStruct((N_NODES[d], H), jnp.float32)
                for (_, r, d) in EDGE_TYPES]
    if with_counts:
        out_type += [jax.ShapeDtypeStruct((N_NODES[d], H), jnp.float32)
                     for (_, r, d) in EDGE_TYPES]
    scratch = [
        pltpu.VMEM_SHARED((SP_ROWS, H), jnp.float32),
        pltpu.VMEM((SLAB, KB), jnp.int32),
        pltpu.VMEM((SLAB, KB), jnp.int32),
        pltpu.VMEM((2 * SLAB + 2, KB), jnp.int32),
        pltpu.VMEM((2 * SLAB + 2, KB), jnp.int32),
        pltpu.VMEM((KB, H), jnp.float32),
        pltpu.VMEM((KB, H), jnp.float32),
        pltpu.VMEM((KB, H), jnp.float32),
        pltpu.SemaphoreType.DMA,
        pltpu.SemaphoreType.DMA,
        pltpu.SemaphoreType.DMA,
    ]
    return pl.kernel(
        functools.partial(_agg_body, with_counts),
        out_type=tuple(out_type),
        mesh=mesh,
        scratch_types=scratch,
        compiler_params=pltpu.CompilerParams(needs_layout_passes=False),
        name=f"sage_agg_counts{int(with_counts)}",
    )


_AGG_WITH_COUNTS = _make_agg(True)
_AGG_NO_COUNTS = _make_agg(False)


def _sc_aggregate(xd, eis_padded, consts, with_counts):
    args = [xd["show"], xd["performance"], xd["song"]]
    for (_, r, _) in EDGE_TYPES:
        args += [eis_padded[r][0], eis_padded[r][1]]
    args += list(consts)
    fn = _AGG_WITH_COUNTS if with_counts else _AGG_NO_COUNTS
    outs = fn(*args)
    s_out = {r: outs[j] for j, (_, r, _) in enumerate(EDGE_TYPES)}
    cnt_out = None
    if with_counts:
        cnt_out = {r: outs[len(EDGE_TYPES) + j]
                   for j, (_, r, _) in enumerate(EDGE_TYPES)}
    return s_out, cnt_out


BLK = 400


def _root_body(x_ref, wrs_ref, bs_ref, out_ref):
    out_ref[...] = lax.dot_general(
        x_ref[...], wrs_ref[...], (((1,), (0,)), ((), ())),
        preferred_element_type=jnp.float32,
        precision=lax.Precision.HIGHEST,
    ) + bs_ref[...]


def _root_layer(x, wrs, bs):
    n = x.shape[0]
    row_spec = pl.BlockSpec((BLK, H), lambda i: (i, 0))
    return pl.pallas_call(
        _root_body,
        grid=(n // BLK,),
        in_specs=[row_spec, pl.BlockSpec((H, H), lambda i: (0, 0)),
                  pl.BlockSpec((1, H), lambda i: (0, 0))],
        out_specs=row_spec,
        out_shape=jax.ShapeDtypeStruct((n, H), jnp.float32),
    )(x, wrs, bs)


def _dense_body(n_r, relu, *refs):
    idx = 0
    s_refs, c_refs = [], []
    for _ in range(n_r):
        s_refs.append(refs[idx]); idx += 1
        c_refs.append(refs[idx]); idx += 1
    root_ref = refs[idx]; idx += 1
    wl_refs = refs[idx:idx + n_r]; idx += n_r
    out_ref = refs[idx]

    acc = root_ref[...]
    for r in range(n_r):
        cnt = jnp.maximum(c_refs[r][...], 1.0)
        agg = s_refs[r][...] / cnt
        acc = acc + lax.dot_general(
            agg, wl_refs[r][...], (((1,), (0,)), ((), ())),
            preferred_element_type=jnp.float32,
            precision=lax.Precision.HIGHEST,
        )
    if relu:
        acc = jnp.maximum(acc, 0.0)
    out_ref[...] = acc


def _dense_layer(n_r, relu, s_list, cnt_list, root, wl_list):
    n = root.shape[0]
    grid = (n // BLK,)
    row_spec = pl.BlockSpec((BLK, H), lambda i: (i, 0))
    cnt_spec = pl.BlockSpec((BLK, 1), lambda i: (i, 0))
    full_spec = pl.BlockSpec((H, H), lambda i: (0, 0))
    in_specs = []
    args = []
    for r in range(n_r):
        in_specs += [row_spec, cnt_spec]
        args += [s_list[r], cnt_list[r]]
    in_specs += [row_spec] + [full_spec] * n_r
    args += [root] + list(wl_list)
    return pl.pallas_call(
        functools.partial(_dense_body, n_r, relu),
        grid=grid,
        in_specs=in_specs,
        out_specs=row_spec,
        out_shape=jax.ShapeDtypeStruct((n, H), jnp.float32),
    )(*args)


def _layer(xd, eis_padded, consts, params, layer, relu, cnt_prev):
    with_counts = cnt_prev is None
    roots = {}
    for d, rels in DST_GROUPS.items():
        wrs = sum(params[f"Wr{layer}_{r}"] for r in rels)
        bs = sum(params[f"b{layer}_{r}"] for r in rels).reshape(1, H)
        roots[d] = _root_layer(xd[d], wrs, bs)
    s_out, cnt_out = _sc_aggregate(xd, eis_padded, consts, with_counts)
    if cnt_out is None:
        cnt_out = cnt_prev
    out = {}
    for d, rels in DST_GROUPS.items():
        out[d] = _dense_layer(
            len(rels), relu,
            [s_out[r] for r in rels],
            [cnt_out[r][:, :1] for r in rels],
            roots[d],
            [params[f"Wl{layer}_{r}"] for r in rels],
        )
    return out, cnt_out


def kernel(x_show, x_performance, x_song, ei_has_performance, ei_of_song, ei_transitioned_to, ei_setlist_neighbor, ei_rev_has_performance, ei_rev_of_song, ei_rev_transitioned_to, Wl1_has_performance, Wr1_has_performance, b1_has_performance, Wl1_of_song, Wr1_of_song, b1_of_song, Wl1_transitioned_to, Wr1_transitioned_to, b1_transitioned_to, Wl1_setlist_neighbor, Wr1_setlist_neighbor, b1_setlist_neighbor, Wl1_rev_has_performance, Wr1_rev_has_performance, b1_rev_has_performance, Wl1_rev_of_song, Wr1_rev_of_song, b1_rev_of_song, Wl1_rev_transitioned_to, Wr1_rev_transitioned_to, b1_rev_transitioned_to, Wl2_has_performance, Wr2_has_performance, b2_has_performance, Wl2_of_song, Wr2_of_song, b2_of_song, Wl2_transitioned_to, Wr2_transitioned_to, b2_transitioned_to, Wl2_setlist_neighbor, Wr2_setlist_neighbor, b2_setlist_neighbor, Wl2_rev_has_performance, Wr2_rev_has_performance, b2_rev_has_performance, Wl2_rev_of_song, Wr2_rev_of_song, b2_rev_of_song, Wl2_rev_transitioned_to, Wr2_rev_transitioned_to, b2_rev_transitioned_to):
    kw = dict(locals())
    params = {k: v for k, v in kw.items()
              if k[:2] in ("Wl", "Wr") or k[0] == "b"}
    xd = {"show": x_show, "performance": x_performance, "song": x_song}

    pad_n = E_PAD - N_EDGES
    eis_padded = {}
    for (_, r, d) in EDGE_TYPES:
        ei = kw[f"ei_{r}"]
        src_p = jnp.concatenate([ei[0], jnp.zeros((pad_n,), ei.dtype)])
        dst_p = jnp.concatenate([ei[1], jnp.full((pad_n,), -1, ei.dtype)])
        eis_padded[r] = (src_p.astype(jnp.int32).reshape(NW, SLAB, 128),
                         dst_p.astype(jnp.int32).reshape(NW, SLAB, 128))

    consts = (
        jnp.zeros((KB, H), jnp.float32),
        jnp.ones((KB, H), jnp.float32),
    )

    h, cnt = _layer(xd, eis_padded, consts, params, 1, True, None)
    h, _ = _layer(h, eis_padded, consts, params, 2, False, cnt)
    return (h["show"], h["performance"], h["song"])


# speedup vs baseline: 1.0010x; 1.0010x over previous
import functools

import jax
import jax.numpy as jnp
from jax import lax
from jax.experimental import pallas as pl
from jax.experimental.pallas import tpu as pltpu
from jax.experimental.pallas import tpu_sc as plsc

H = 128
N_NODES = {"show": 10000, "performance": 100000, "song": 10000}
EDGE_TYPES = [
    ("show", "has_performance", "performance"),
    ("performance", "of_song", "song"),
    ("song", "transitioned_to", "song"),
    ("show", "setlist_neighbor", "show"),
    ("performance", "rev_has_performance", "show"),
    ("song", "rev_of_song", "performance"),
    ("song", "rev_transitioned_to", "song"),
]
N_EDGES = 100000

DST_GROUPS = {
    "show": ["setlist_neighbor", "rev_has_performance"],
    "performance": ["has_performance", "rev_of_song"],
    "song": ["of_song", "transitioned_to", "rev_transitioned_to"],
}

NC, NS = 2, 16
NW = NC * NS
E_PAD = 102400
EB = E_PAD // 128
SLAB = EB // NW
KB = 128
CHUNK = 5000
SP_ROWS = 5120
ZTR = 320


def _agg_body(with_counts, *refs):
    i = 0
    x_refs = {"show": refs[0], "performance": refs[1], "song": refs[2]}
    i = 3
    ei_refs = {}
    for (_, r, _) in EDGE_TYPES:
        ei_refs[r] = (refs[i], refs[i + 1])
        i += 2
    zb_hbm, ones_tab = refs[i], refs[i + 1]
    i += 2
    s_out = {}
    for (_, r, _) in EDGE_TYPES:
        s_out[r] = refs[i]
        i += 1
    c_out = {}
    if with_counts:
        for (_, r, _) in EDGE_TYPES:
            c_out[r] = refs[i]
            i += 1
    (S_sp, src_ids, dst_ids, csrc, cdst, rows, rows2, rows3,
     semA, semB, semC, semSA, semSB, semSC) = refs[i:]

    c = lax.axis_index("c")
    s = lax.axis_index("s")

    def zero_spmem():
        pltpu.sync_copy(zb_hbm, rows)
        r0 = s * ZTR
        for j in range(2):
            pltpu.sync_copy(rows, S_sp.at[pl.ds(r0 + j * KB, KB)])
        pltpu.sync_copy(rows.at[pl.ds(0, 64)], S_sp.at[pl.ds(r0 + 256, 64)])


    def wb(dst_s):
        @pl.when(s < NS - 1)
        def _wb_main():
            r0 = s * 312
            pltpu.sync_copy(S_sp.at[pl.ds(r0, 312)], dst_s(r0, 312))

        @pl.when(s == NS - 1)
        def _wb_tail():
            pltpu.sync_copy(S_sp.at[pl.ds(4680, 320)], dst_s(4680, 320))

    def agg_type(r, x_hbm, n_dst, out_ref, counts_mode):
        src_ref, dst_ref = ei_refs[r]
        iot = lax.iota(jnp.int32, 16)
        ncs = n_dst // (NC * CHUNK)

        def chunk_body(ch, _carry):
            lo = (c * ncs + ch) * CHUNK
            hi = lo + CHUNK
            zero_spmem()
            plsc.subcore_barrier()
            k = 0
            for hh in range(2):
                pltpu.sync_copy(src_ref.at[2 * s + hh], src_ids)
                pltpu.sync_copy(dst_ref.at[2 * s + hh], dst_ids)

                def comp(i2, kk):
                    rr = i2 // 8
                    cc = (i2 % 8) * 16
                    vd = dst_ids[rr, pl.ds(cc, 16)]
                    vs = src_ids[rr, pl.ds(cc, 16)]
                    m = (vd >= lo) & (vd < hi)
                    mi = m.astype(jnp.int32)
                    pos = kk + plsc.cumsum(mi) - mi
                    if not counts_mode:
                        plsc.store_scatter(csrc, [pos // KB, pos % KB], vs,
                                           mask=m)
                    plsc.store_scatter(cdst, [pos // KB, pos % KB], vd - lo,
                                       mask=m)
                    return kk + jnp.sum(mi)

                k = lax.fori_loop(0, SLAB * 8, comp, k)
            for t in range(8):
                pos = k + 16 * t + iot
                if not counts_mode:
                    plsc.store_scatter(csrc, [pos // KB, pos % KB],
                                       jnp.zeros((16,), jnp.int32))
                plsc.store_scatter(cdst, [pos // KB, pos % KB],
                                   jnp.full((16,), CHUNK, jnp.int32))
            nblk = (k + KB - 1) // KB

            if counts_mode:
                pltpu.sync_copy(ones_tab, rows)

                def blk(b, carry):
                    pltpu.sync_copy(rows, S_sp.at[cdst.at[b]], add=True)
                    return carry

                lax.fori_loop(0, nblk, blk, 0)
            else:
                bufs = ((rows, semA, semSA), (rows2, semB, semSB),
                        (rows3, semC, semSC))
                pltpu.async_copy(x_hbm.at[csrc.at[0]], rows, semA)
                for j in (1, 2):
                    @pl.when(j < nblk)
                    def _gp(j=j):
                        pltpu.async_copy(x_hbm.at[csrc.at[j]], bufs[j][0],
                                         bufs[j][1])

                def trip(p, carry):
                    b0 = 3 * p
                    for j in range(3):
                        bj, (buf, sj, ssj) = b0 + j, bufs[j]

                        @pl.when(bj < nblk)
                        def _do(bj=bj, buf=buf, sj=sj, ssj=ssj):
                            pltpu.make_async_copy(
                                x_hbm.at[csrc.at[bj]], buf, sj).wait()
                            pltpu.async_copy(buf, S_sp.at[cdst.at[bj]], ssj,
                                             add=True)

                            @pl.when(bj + 3 < nblk)
                            def _gn():
                                pltpu.make_async_copy(zb_hbm, buf, ssj).wait()
                                pltpu.async_copy(
                                    x_hbm.at[csrc.at[bj + 3]], buf, sj)

                    return carry

                lax.fori_loop(0, (nblk + 2) // 3, trip, 0)
                for j in range(3):
                    @pl.when(jnp.maximum(nblk - 3, 0) + j < nblk)
                    def _dr(j=j):
                        pltpu.make_async_copy(zb_hbm, bufs[j][0],
                                              bufs[j][2]).wait()
            plsc.subcore_barrier()
            wb(lambda r0, n: out_ref.at[pl.ds(lo + r0, n)])
            plsc.subcore_barrier()
            return _carry

        lax.fori_loop(0, ncs, chunk_body, 0)

    for (st, r, d) in EDGE_TYPES:
        agg_type(r, x_refs[st], N_NODES[d], s_out[r], False)
        if with_counts:
            agg_type(r, ones_tab, N_NODES[d], c_out[r], True)


def _make_agg(with_counts):
    mesh = plsc.VectorSubcoreMesh(
        core_axis_name="c", subcore_axis_name="s", num_cores=NC, num_subcores=NS)
    out_type = [jax.ShapeDtypeStruct((N_NODES[d], H), jnp.float32)
                for (_, r, d) in EDGE_TYPES]
    if with_counts:
        out_type += [jax.ShapeDtypeStruct((N_NODES[d], H), jnp.float32)
                     for (_, r, d) in EDGE_TYPES]
    scratch = [
        pltpu.VMEM_SHARED((SP_ROWS, H), jnp.float32),
        pltpu.VMEM((SLAB, KB), jnp.int32),
        pltpu.VMEM((SLAB, KB), jnp.int32),
        pltpu.VMEM((2 * SLAB + 2, KB), jnp.int32),
        pltpu.VMEM((2 * SLAB + 2, KB), jnp.int32),
        pltpu.VMEM((KB, H), jnp.float32),
        pltpu.VMEM((KB, H), jnp.float32),
        pltpu.VMEM((KB, H), jnp.float32),
        pltpu.SemaphoreType.DMA,
        pltpu.SemaphoreType.DMA,
        pltpu.SemaphoreType.DMA,
        pltpu.SemaphoreType.DMA,
        pltpu.SemaphoreType.DMA,
        pltpu.SemaphoreType.DMA,
    ]
    return pl.kernel(
        functools.partial(_agg_body, with_counts),
        out_type=tuple(out_type),
        mesh=mesh,
        scratch_types=scratch,
        compiler_params=pltpu.CompilerParams(needs_layout_passes=False),
        name=f"sage_agg_counts{int(with_counts)}",
    )


_AGG_WITH_COUNTS = _make_agg(True)
_AGG_NO_COUNTS = _make_agg(False)


def _sc_aggregate(xd, eis_padded, consts, with_counts):
    args = [xd["show"], xd["performance"], xd["song"]]
    for (_, r, _) in EDGE_TYPES:
        args += [eis_padded[r][0], eis_padded[r][1]]
    args += list(consts)
    fn = _AGG_WITH_COUNTS if with_counts else _AGG_NO_COUNTS
    outs = fn(*args)
    s_out = {r: outs[j] for j, (_, r, _) in enumerate(EDGE_TYPES)}
    cnt_out = None
    if with_counts:
        cnt_out = {r: outs[len(EDGE_TYPES) + j]
                   for j, (_, r, _) in enumerate(EDGE_TYPES)}
    return s_out, cnt_out


BLK = 400


def _root_body(x_ref, wrs_ref, bs_ref, out_ref):
    out_ref[...] = lax.dot_general(
        x_ref[...], wrs_ref[...], (((1,), (0,)), ((), ())),
        preferred_element_type=jnp.float32,
        precision=lax.Precision.HIGHEST,
    ) + bs_ref[...]


def _root_layer(x, wrs, bs):
    n = x.shape[0]
    row_spec = pl.BlockSpec((BLK, H), lambda i: (i, 0))
    return pl.pallas_call(
        _root_body,
        grid=(n // BLK,),
        in_specs=[row_spec, pl.BlockSpec((H, H), lambda i: (0, 0)),
                  pl.BlockSpec((1, H), lambda i: (0, 0))],
        out_specs=row_spec,
        out_shape=jax.ShapeDtypeStruct((n, H), jnp.float32),
    )(x, wrs, bs)


def _dense_body(n_r, relu, *refs):
    idx = 0
    s_refs, c_refs = [], []
    for _ in range(n_r):
        s_refs.append(refs[idx]); idx += 1
        c_refs.append(refs[idx]); idx += 1
    root_ref = refs[idx]; idx += 1
    wl_refs = refs[idx:idx + n_r]; idx += n_r
    out_ref = refs[idx]

    acc = root_ref[...]
    for r in range(n_r):
        cnt = jnp.maximum(c_refs[r][...], 1.0)
        agg = s_refs[r][...] / cnt
        acc = acc + lax.dot_general(
            agg, wl_refs[r][...], (((1,), (0,)), ((), ())),
            preferred_element_type=jnp.float32,
            precision=lax.Precision.HIGHEST,
        )
    if relu:
        acc = jnp.maximum(acc, 0.0)
    out_ref[...] = acc


def _dense_layer(n_r, relu, s_list, cnt_list, root, wl_list):
    n = root.shape[0]
    grid = (n // BLK,)
    row_spec = pl.BlockSpec((BLK, H), lambda i: (i, 0))
    cnt_spec = pl.BlockSpec((BLK, 1), lambda i: (i, 0))
    full_spec = pl.BlockSpec((H, H), lambda i: (0, 0))
    in_specs = []
    args = []
    for r in range(n_r):
        in_specs += [row_spec, cnt_spec]
        args += [s_list[r], cnt_list[r]]
    in_specs += [row_spec] + [full_spec] * n_r
    args += [root] + list(wl_list)
    return pl.pallas_call(
        functools.partial(_dense_body, n_r, relu),
        grid=grid,
        in_specs=in_specs,
        out_specs=row_spec,
        out_shape=jax.ShapeDtypeStruct((n, H), jnp.float32),
    )(*args)


def _layer(xd, eis_padded, consts, params, layer, relu, cnt_prev):
    with_counts = cnt_prev is None
    roots = {}
    for d, rels in DST_GROUPS.items():
        wrs = sum(params[f"Wr{layer}_{r}"] for r in rels)
        bs = sum(params[f"b{layer}_{r}"] for r in rels).reshape(1, H)
        roots[d] = _root_layer(xd[d], wrs, bs)
    s_out, cnt_out = _sc_aggregate(xd, eis_padded, consts, with_counts)
    if cnt_out is None:
        cnt_out = cnt_prev
    out = {}
    for d, rels in DST_GROUPS.items():
        out[d] = _dense_layer(
            len(rels), relu,
            [s_out[r] for r in rels],
            [cnt_out[r][:, :1] for r in rels],
            roots[d],
            [params[f"Wl{layer}_{r}"] for r in rels],
        )
    return out, cnt_out


def kernel(x_show, x_performance, x_song, ei_has_performance, ei_of_song, ei_transitioned_to, ei_setlist_neighbor, ei_rev_has_performance, ei_rev_of_song, ei_rev_transitioned_to, Wl1_has_performance, Wr1_has_performance, b1_has_performance, Wl1_of_song, Wr1_of_song, b1_of_song, Wl1_transitioned_to, Wr1_transitioned_to, b1_transitioned_to, Wl1_setlist_neighbor, Wr1_setlist_neighbor, b1_setlist_neighbor, Wl1_rev_has_performance, Wr1_rev_has_performance, b1_rev_has_performance, Wl1_rev_of_song, Wr1_rev_of_song, b1_rev_of_song, Wl1_rev_transitioned_to, Wr1_rev_transitioned_to, b1_rev_transitioned_to, Wl2_has_performance, Wr2_has_performance, b2_has_performance, Wl2_of_song, Wr2_of_song, b2_of_song, Wl2_transitioned_to, Wr2_transitioned_to, b2_transitioned_to, Wl2_setlist_neighbor, Wr2_setlist_neighbor, b2_setlist_neighbor, Wl2_rev_has_performance, Wr2_rev_has_performance, b2_rev_has_performance, Wl2_rev_of_song, Wr2_rev_of_song, b2_rev_of_song, Wl2_rev_transitioned_to, Wr2_rev_transitioned_to, b2_rev_transitioned_to):
    kw = dict(locals())
    params = {k: v for k, v in kw.items()
              if k[:2] in ("Wl", "Wr") or k[0] == "b"}
    xd = {"show": x_show, "performance": x_performance, "song": x_song}

    pad_n = E_PAD - N_EDGES
    eis_padded = {}
    for (_, r, d) in EDGE_TYPES:
        ei = kw[f"ei_{r}"]
        src_p = jnp.concatenate([ei[0], jnp.zeros((pad_n,), ei.dtype)])
        dst_p = jnp.concatenate([ei[1], jnp.full((pad_n,), -1, ei.dtype)])
        eis_padded[r] = (src_p.astype(jnp.int32).reshape(NW, SLAB, 128),
                         dst_p.astype(jnp.int32).reshape(NW, SLAB, 128))

    consts = (
        jnp.zeros((KB, H), jnp.float32),
        jnp.ones((KB, H), jnp.float32),
    )

    h, cnt = _layer(xd, eis_padded, consts, params, 1, True, None)
    h, _ = _layer(h, eis_padded, consts, params, 2, False, cnt)
    return (h["show"], h["performance"], h["song"])


# speedup vs baseline: 1.0353x; 1.0342x over previous
import functools

import jax
import jax.numpy as jnp
from jax import lax
from jax.experimental import pallas as pl
from jax.experimental.pallas import tpu as pltpu
from jax.experimental.pallas import tpu_sc as plsc

H = 128
N_NODES = {"show": 10000, "performance": 100000, "song": 10000}
EDGE_TYPES = [
    ("show", "has_performance", "performance"),
    ("performance", "of_song", "song"),
    ("song", "transitioned_to", "song"),
    ("show", "setlist_neighbor", "show"),
    ("performance", "rev_has_performance", "show"),
    ("song", "rev_of_song", "performance"),
    ("song", "rev_transitioned_to", "song"),
]
N_EDGES = 100000

DST_GROUPS = {
    "show": ["setlist_neighbor", "rev_has_performance"],
    "performance": ["has_performance", "rev_of_song"],
    "song": ["of_song", "transitioned_to", "rev_transitioned_to"],
}

NC, NS = 2, 16
NW = NC * NS
E_PAD = 102400
EB = E_PAD // 128
SLAB = EB // NW
KB = 128
CHUNK = 5000
SP_ROWS = 5120
ZTR = 320


def _agg_body(with_counts, *refs):
    i = 0
    x_refs = {"show": refs[0], "performance": refs[1], "song": refs[2]}
    i = 3
    ei_refs = {}
    for (_, r, _) in EDGE_TYPES:
        ei_refs[r] = (refs[i], refs[i + 1])
        i += 2
    zb_hbm, ones_tab = refs[i], refs[i + 1]
    i += 2
    s_out = {}
    for (_, r, _) in EDGE_TYPES:
        s_out[r] = refs[i]
        i += 1
    c_out = {}
    if with_counts:
        for (_, r, _) in EDGE_TYPES:
            c_out[r] = refs[i]
            i += 1
    (S_sp, src_ids, dst_ids, csrc, cdst, rows, rows2, zbuf,
     semA, semB) = refs[i:]

    c = lax.axis_index("c")
    s = lax.axis_index("s")

    pltpu.sync_copy(zb_hbm, zbuf)

    def zero_spmem():
        r0 = s * ZTR
        for j in range(2):
            pltpu.sync_copy(zbuf, S_sp.at[pl.ds(r0 + j * KB, KB)])
        pltpu.sync_copy(zbuf.at[pl.ds(0, 64)], S_sp.at[pl.ds(r0 + 256, 64)])


    def wb(dst_s):
        @pl.when(s < NS - 1)
        def _wb_main():
            r0 = s * 312
            pltpu.sync_copy(S_sp.at[pl.ds(r0, 312)], dst_s(r0, 312))

        @pl.when(s == NS - 1)
        def _wb_tail():
            pltpu.sync_copy(S_sp.at[pl.ds(4680, 320)], dst_s(4680, 320))

    def agg_type(r, x_hbm, n_dst, out_ref, counts_mode):
        src_ref, dst_ref = ei_refs[r]
        iot = lax.iota(jnp.int32, 16)
        ncs = n_dst // (NC * CHUNK)
        if counts_mode:
            pltpu.sync_copy(ones_tab, rows2)

        def chunk_body(ch, _carry):
            lo = (c * ncs + ch) * CHUNK
            hi = lo + CHUNK
            zero_spmem()
            plsc.subcore_barrier()
            k = 0
            for hh in range(2):
                pltpu.sync_copy(src_ref.at[2 * s + hh], src_ids)
                pltpu.sync_copy(dst_ref.at[2 * s + hh], dst_ids)

                def comp(i2, kk):
                    rr = i2 // 8
                    cc = (i2 % 8) * 16
                    vd = dst_ids[rr, pl.ds(cc, 16)]
                    vs = src_ids[rr, pl.ds(cc, 16)]
                    m = (vd >= lo) & (vd < hi)
                    mi = m.astype(jnp.int32)
                    pos = kk + plsc.cumsum(mi) - mi
                    if not counts_mode:
                        plsc.store_scatter(csrc, [pos // KB, pos % KB], vs,
                                           mask=m)
                    plsc.store_scatter(cdst, [pos // KB, pos % KB], vd - lo,
                                       mask=m)
                    return kk + jnp.sum(mi)

                k = lax.fori_loop(0, SLAB * 8, comp, k)
            for t in range(8):
                pos = k + 16 * t + iot
                if not counts_mode:
                    plsc.store_scatter(csrc, [pos // KB, pos % KB],
                                       jnp.zeros((16,), jnp.int32))
                plsc.store_scatter(cdst, [pos // KB, pos % KB],
                                   jnp.full((16,), CHUNK, jnp.int32))
            nblk = (k + KB - 1) // KB

            if counts_mode:
                def blk(b, carry):
                    pltpu.sync_copy(rows2, S_sp.at[cdst.at[b]], add=True)
                    return carry

                lax.fori_loop(0, nblk, blk, 0)
            else:
                pltpu.async_copy(x_hbm.at[csrc.at[0]], rows, semA)

                def pair(p, carry):
                    b0 = 2 * p

                    @pl.when(b0 + 1 < nblk)
                    def _g1():
                        pltpu.async_copy(x_hbm.at[csrc.at[b0 + 1]], rows2, semB)

                    pltpu.make_async_copy(x_hbm.at[csrc.at[b0]], rows, semA).wait()
                    pltpu.sync_copy(rows, S_sp.at[cdst.at[b0]], add=True)

                    @pl.when(b0 + 2 < nblk)
                    def _g2():
                        pltpu.async_copy(x_hbm.at[csrc.at[b0 + 2]], rows, semA)

                    @pl.when(b0 + 1 < nblk)
                    def _s1():
                        pltpu.make_async_copy(
                            x_hbm.at[csrc.at[b0 + 1]], rows2, semB).wait()
                        pltpu.sync_copy(rows2, S_sp.at[cdst.at[b0 + 1]], add=True)

                    return carry

                lax.fori_loop(0, (nblk + 1) // 2, pair, 0)
            plsc.subcore_barrier()
            wb(lambda r0, n: out_ref.at[pl.ds(lo + r0, n)])
            plsc.subcore_barrier()
            return _carry

        lax.fori_loop(0, ncs, chunk_body, 0)

    for (st, r, d) in EDGE_TYPES:
        agg_type(r, x_refs[st], N_NODES[d], s_out[r], False)
        if with_counts:
            agg_type(r, ones_tab, N_NODES[d], c_out[r], True)


def _make_agg(with_counts):
    mesh = plsc.VectorSubcoreMesh(
        core_axis_name="c", subcore_axis_name="s", num_cores=NC, num_subcores=NS)
    out_type = [jax.ShapeDtypeStruct((N_NODES[d], H), jnp.float32)
                for (_, r, d) in EDGE_TYPES]
    if with_counts:
        out_type += [jax.ShapeDtypeStruct((N_NODES[d], H), jnp.float32)
                     for (_, r, d) in EDGE_TYPES]
    scratch = [
        pltpu.VMEM_SHARED((SP_ROWS, H), jnp.float32),
        pltpu.VMEM((SLAB, KB), jnp.int32),
        pltpu.VMEM((SLAB, KB), jnp.int32),
        pltpu.VMEM((2 * SLAB + 2, KB), jnp.int32),
        pltpu.VMEM((2 * SLAB + 2, KB), jnp.int32),
        pltpu.VMEM((KB, H), jnp.float32),
        pltpu.VMEM((KB, H), jnp.float32),
        pltpu.VMEM((KB, H), jnp.float32),
        pltpu.SemaphoreType.DMA,
        pltpu.SemaphoreType.DMA,
    ]
    return pl.kernel(
        functools.partial(_agg_body, with_counts),
        out_type=tuple(out_type),
        mesh=mesh,
        scratch_types=scratch,
        compiler_params=pltpu.CompilerParams(needs_layout_passes=False),
        name=f"sage_agg_counts{int(with_counts)}",
    )


_AGG_WITH_COUNTS = _make_agg(True)
_AGG_NO_COUNTS = _make_agg(False)


def _sc_aggregate(xd, eis_padded, consts, with_counts):
    args = [xd["show"], xd["performance"], xd["song"]]
    for (_, r, _) in EDGE_TYPES:
        args += [eis_padded[r][0], eis_padded[r][1]]
    args += list(consts)
    fn = _AGG_WITH_COUNTS if with_counts else _AGG_NO_COUNTS
    outs = fn(*args)
    s_out = {r: outs[j] for j, (_, r, _) in enumerate(EDGE_TYPES)}
    cnt_out = None
    if with_counts:
        cnt_out = {r: outs[len(EDGE_TYPES) + j]
                   for j, (_, r, _) in enumerate(EDGE_TYPES)}
    return s_out, cnt_out


BLK = 400


def _root_body(x_ref, wrs_ref, bs_ref, out_ref):
    out_ref[...] = lax.dot_general(
        x_ref[...], wrs_ref[...], (((1,), (0,)), ((), ())),
        preferred_element_type=jnp.float32,
        precision=lax.Precision.HIGHEST,
    ) + bs_ref[...]


def _root_layer(x, wrs, bs):
    n = x.shape[0]
    row_spec = pl.BlockSpec((BLK, H), lambda i: (i, 0))
    return pl.pallas_call(
        _root_body,
        grid=(n // BLK,),
        in_specs=[row_spec, pl.BlockSpec((H, H), lambda i: (0, 0)),
                  pl.BlockSpec((1, H), lambda i: (0, 0))],
        out_specs=row_spec,
        out_shape=jax.ShapeDtypeStruct((n, H), jnp.float32),
    )(x, wrs, bs)


def _dense_body(n_r, relu, *refs):
    idx = 0
    s_refs, c_refs = [], []
    for _ in range(n_r):
        s_refs.append(refs[idx]); idx += 1
        c_refs.append(refs[idx]); idx += 1
    root_ref = refs[idx]; idx += 1
    wl_refs = refs[idx:idx + n_r]; idx += n_r
    out_ref = refs[idx]

    acc = root_ref[...]
    for r in range(n_r):
        cnt = jnp.maximum(c_refs[r][...], 1.0)
        agg = s_refs[r][...] / cnt
        acc = acc + lax.dot_general(
            agg, wl_refs[r][...], (((1,), (0,)), ((), ())),
            preferred_element_type=jnp.float32,
            precision=lax.Precision.HIGHEST,
        )
    if relu:
        acc = jnp.maximum(acc, 0.0)
    out_ref[...] = acc


def _dense_layer(n_r, relu, s_list, cnt_list, root, wl_list):
    n = root.shape[0]
    grid = (n // BLK,)
    row_spec = pl.BlockSpec((BLK, H), lambda i: (i, 0))
    cnt_spec = pl.BlockSpec((BLK, 1), lambda i: (i, 0))
    full_spec = pl.BlockSpec((H, H), lambda i: (0, 0))
    in_specs = []
    args = []
    for r in range(n_r):
        in_specs += [row_spec, cnt_spec]
        args += [s_list[r], cnt_list[r]]
    in_specs += [row_spec] + [full_spec] * n_r
    args += [root] + list(wl_list)
    return pl.pallas_call(
        functools.partial(_dense_body, n_r, relu),
        grid=grid,
        in_specs=in_specs,
        out_specs=row_spec,
        out_shape=jax.ShapeDtypeStruct((n, H), jnp.float32),
    )(*args)


def _layer(xd, eis_padded, consts, params, layer, relu, cnt_prev):
    with_counts = cnt_prev is None
    roots = {}
    for d, rels in DST_GROUPS.items():
        wrs = sum(params[f"Wr{layer}_{r}"] for r in rels)
        bs = sum(params[f"b{layer}_{r}"] for r in rels).reshape(1, H)
        roots[d] = _root_layer(xd[d], wrs, bs)
    s_out, cnt_out = _sc_aggregate(xd, eis_padded, consts, with_counts)
    if cnt_out is None:
        cnt_out = cnt_prev
    out = {}
    for d, rels in DST_GROUPS.items():
        out[d] = _dense_layer(
            len(rels), relu,
            [s_out[r] for r in rels],
            [cnt_out[r][:, :1] for r in rels],
            roots[d],
            [params[f"Wl{layer}_{r}"] for r in rels],
        )
    return out, cnt_out


def kernel(x_show, x_performance, x_song, ei_has_performance, ei_of_song, ei_transitioned_to, ei_setlist_neighbor, ei_rev_has_performance, ei_rev_of_song, ei_rev_transitioned_to, Wl1_has_performance, Wr1_has_performance, b1_has_performance, Wl1_of_song, Wr1_of_song, b1_of_song, Wl1_transitioned_to, Wr1_transitioned_to, b1_transitioned_to, Wl1_setlist_neighbor, Wr1_setlist_neighbor, b1_setlist_neighbor, Wl1_rev_has_performance, Wr1_rev_has_performance, b1_rev_has_performance, Wl1_rev_of_song, Wr1_rev_of_song, b1_rev_of_song, Wl1_rev_transitioned_to, Wr1_rev_transitioned_to, b1_rev_transitioned_to, Wl2_has_performance, Wr2_has_performance, b2_has_performance, Wl2_of_song, Wr2_of_song, b2_of_song, Wl2_transitioned_to, Wr2_transitioned_to, b2_transitioned_to, Wl2_setlist_neighbor, Wr2_setlist_neighbor, b2_setlist_neighbor, Wl2_rev_has_performance, Wr2_rev_has_performance, b2_rev_has_performance, Wl2_rev_of_song, Wr2_rev_of_song, b2_rev_of_song, Wl2_rev_transitioned_to, Wr2_rev_transitioned_to, b2_rev_transitioned_to):
    kw = dict(locals())
    params = {k: v for k, v in kw.items()
              if k[:2] in ("Wl", "Wr") or k[0] == "b"}
    xd = {"show": x_show, "performance": x_performance, "song": x_song}

    pad_n = E_PAD - N_EDGES
    eis_padded = {}
    for (_, r, d) in EDGE_TYPES:
        ei = kw[f"ei_{r}"]
        src_p = jnp.concatenate([ei[0], jnp.zeros((pad_n,), ei.dtype)])
        dst_p = jnp.concatenate([ei[1], jnp.full((pad_n,), -1, ei.dtype)])
        eis_padded[r] = (src_p.astype(jnp.int32).reshape(NW, SLAB, 128),
                         dst_p.astype(jnp.int32).reshape(NW, SLAB, 128))

    consts = (
        jnp.zeros((KB, H), jnp.float32),
        jnp.ones((KB, H), jnp.float32),
    )

    h, cnt = _layer(xd, eis_padded, consts, params, 1, True, None)
    h, _ = _layer(h, eis_padded, consts, params, 2, False, cnt)
    return (h["show"], h["performance"], h["song"])


# speedup vs baseline: 2.7751x; 2.6806x over previous
import functools

import jax
import jax.numpy as jnp
from jax import lax
from jax.experimental import pallas as pl
from jax.experimental.pallas import tpu as pltpu
from jax.experimental.pallas import tpu_sc as plsc

H = 128
N_NODES = {"show": 10000, "performance": 100000, "song": 10000}
EDGE_TYPES = [
    ("show", "has_performance", "performance"),
    ("performance", "of_song", "song"),
    ("song", "transitioned_to", "song"),
    ("show", "setlist_neighbor", "show"),
    ("performance", "rev_has_performance", "show"),
    ("song", "rev_of_song", "performance"),
    ("song", "rev_transitioned_to", "song"),
]
N_EDGES = 100000

DST_GROUPS = {
    "show": ["setlist_neighbor", "rev_has_performance"],
    "performance": ["has_performance", "rev_of_song"],
    "song": ["of_song", "transitioned_to", "rev_transitioned_to"],
}

NC, NS = 2, 16
NW = NC * NS
E_PAD = 102400
EB = E_PAD // 128
SLAB = EB // NW
KB = 128
CHUNK = 5000
SP_ROWS = 5120
ZTR = 320


def _agg_body(with_counts, *refs):
    i = 0
    x_refs = {"show": refs[0], "performance": refs[1], "song": refs[2]}
    i = 3
    ei_refs = {}
    for (_, r, _) in EDGE_TYPES:
        ei_refs[r] = (refs[i], refs[i + 1])
        i += 2
    zb_hbm, ones_tab = refs[i], refs[i + 1]
    i += 2
    s_out = {}
    for (_, r, _) in EDGE_TYPES:
        s_out[r] = refs[i]
        i += 1
    c_out = {}
    if with_counts:
        for (_, r, _) in EDGE_TYPES:
            c_out[r] = refs[i]
            i += 1
    (S_sp, src_ids, dst_ids, csrc, cdst, rows, rows2, zbuf,
     semA, semB) = refs[i:]

    c = lax.axis_index("c")
    s = lax.axis_index("s")

    pltpu.sync_copy(zb_hbm, zbuf)

    def zero_spmem():
        r0 = s * ZTR
        for j in range(2):
            pltpu.sync_copy(zbuf, S_sp.at[pl.ds(r0 + j * KB, KB)])
        pltpu.sync_copy(zbuf.at[pl.ds(0, 64)], S_sp.at[pl.ds(r0 + 256, 64)])


    def wb(dst_s):
        @pl.when(s < NS - 1)
        def _wb_main():
            r0 = s * 312
            pltpu.sync_copy(S_sp.at[pl.ds(r0, 312)], dst_s(r0, 312))

        @pl.when(s == NS - 1)
        def _wb_tail():
            pltpu.sync_copy(S_sp.at[pl.ds(4680, 320)], dst_s(4680, 320))

    def agg_type(r, x_hbm, n_dst, out_s, out_c):
        src_ref, dst_ref = ei_refs[r]
        iot = lax.iota(jnp.int32, 16)
        ncs = n_dst // (NC * CHUNK)

        def chunk_body(ch, _carry):
            lo = (c * ncs + ch) * CHUNK
            hi = lo + CHUNK
            zero_spmem()
            plsc.subcore_barrier()
            k = 0
            for hh in range(2):
                pltpu.sync_copy(src_ref.at[2 * s + hh], src_ids)
                pltpu.sync_copy(dst_ref.at[2 * s + hh], dst_ids)

                def comp(i2, kk):
                    rr = i2 // 8
                    cc = (i2 % 8) * 16
                    vd = dst_ids[rr, pl.ds(cc, 16)]
                    vs = src_ids[rr, pl.ds(cc, 16)]
                    m = (vd >= lo) & (vd < hi)
                    mi = m.astype(jnp.int32)
                    pos = kk + plsc.cumsum(mi) - mi
                    plsc.store_scatter(csrc, [pos // KB, pos % KB], vs, mask=m)
                    plsc.store_scatter(cdst, [pos // KB, pos % KB], vd - lo,
                                       mask=m)
                    return kk + jnp.sum(mi)

                k = lax.fori_loop(0, SLAB * 8, comp, k)
            for t in range(8):
                pos = k + 16 * t + iot
                plsc.store_scatter(csrc, [pos // KB, pos % KB], pos)
                plsc.store_scatter(cdst, [pos // KB, pos % KB],
                                   jnp.full((16,), CHUNK, jnp.int32))
            nblk = (k + KB - 1) // KB

            pltpu.async_copy(x_hbm.at[csrc.at[0]], rows, semA)

            def pair(p, carry):
                b0 = 2 * p

                @pl.when(b0 + 1 < nblk)
                def _g1():
                    pltpu.async_copy(x_hbm.at[csrc.at[b0 + 1]], rows2, semB)

                pltpu.make_async_copy(x_hbm.at[csrc.at[b0]], rows, semA).wait()
                pltpu.sync_copy(rows, S_sp.at[cdst.at[b0]], add=True)

                @pl.when(b0 + 2 < nblk)
                def _g2():
                    pltpu.async_copy(x_hbm.at[csrc.at[b0 + 2]], rows, semA)

                @pl.when(b0 + 1 < nblk)
                def _s1():
                    pltpu.make_async_copy(
                        x_hbm.at[csrc.at[b0 + 1]], rows2, semB).wait()
                    pltpu.sync_copy(rows2, S_sp.at[cdst.at[b0 + 1]], add=True)

                return carry

            lax.fori_loop(0, (nblk + 1) // 2, pair, 0)
            plsc.subcore_barrier()
            wb(lambda r0, n: out_s.at[pl.ds(lo + r0, n)])
            plsc.subcore_barrier()

            if out_c is not None:
                zero_spmem()
                pltpu.sync_copy(ones_tab, rows2)
                plsc.subcore_barrier()

                def blk(b, carry):
                    pltpu.sync_copy(rows2, S_sp.at[cdst.at[b]], add=True)
                    return carry

                lax.fori_loop(0, nblk, blk, 0)
                plsc.subcore_barrier()
                wb(lambda r0, n: out_c.at[pl.ds(lo + r0, n)])
                plsc.subcore_barrier()
            return _carry

        lax.fori_loop(0, ncs, chunk_body, 0)

    for (st, r, d) in EDGE_TYPES:
        agg_type(r, x_refs[st], N_NODES[d], s_out[r],
                 c_out[r] if with_counts else None)


def _make_agg(with_counts):
    mesh = plsc.VectorSubcoreMesh(
        core_axis_name="c", subcore_axis_name="s", num_cores=NC, num_subcores=NS)
    out_type = [jax.ShapeDtypeStruct((N_NODES[d], H), jnp.float32)
                for (_, r, d) in EDGE_TYPES]
    if with_counts:
        out_type += [jax.ShapeDtypeStruct((N_NODES[d], H), jnp.float32)
                     for (_, r, d) in EDGE_TYPES]
    scratch = [
        pltpu.VMEM_SHARED((SP_ROWS, H), jnp.float32),
        pltpu.VMEM((SLAB, KB), jnp.int32),
        pltpu.VMEM((SLAB, KB), jnp.int32),
        pltpu.VMEM((2 * SLAB + 2, KB), jnp.int32),
        pltpu.VMEM((2 * SLAB + 2, KB), jnp.int32),
        pltpu.VMEM((KB, H), jnp.float32),
        pltpu.VMEM((KB, H), jnp.float32),
        pltpu.VMEM((KB, H), jnp.float32),
        pltpu.SemaphoreType.DMA,
        pltpu.SemaphoreType.DMA,
    ]
    return pl.kernel(
        functools.partial(_agg_body, with_counts),
        out_type=tuple(out_type),
        mesh=mesh,
        scratch_types=scratch,
        compiler_params=pltpu.CompilerParams(needs_layout_passes=False),
        name=f"sage_agg_counts{int(with_counts)}",
    )


_AGG_WITH_COUNTS = _make_agg(True)
_AGG_NO_COUNTS = _make_agg(False)


def _sc_aggregate(xd, eis_padded, consts, with_counts):
    args = [xd["show"], xd["performance"], xd["song"]]
    for (_, r, _) in EDGE_TYPES:
        args += [eis_padded[r][0], eis_padded[r][1]]
    args += list(consts)
    fn = _AGG_WITH_COUNTS if with_counts else _AGG_NO_COUNTS
    outs = fn(*args)
    s_out = {r: outs[j] for j, (_, r, _) in enumerate(EDGE_TYPES)}
    cnt_out = None
    if with_counts:
        cnt_out = {r: outs[len(EDGE_TYPES) + j]
                   for j, (_, r, _) in enumerate(EDGE_TYPES)}
    return s_out, cnt_out


BLK = 400


def _root_body(x_ref, wrs_ref, bs_ref, out_ref):
    out_ref[...] = lax.dot_general(
        x_ref[...], wrs_ref[...], (((1,), (0,)), ((), ())),
        preferred_element_type=jnp.float32,
        precision=lax.Precision.HIGHEST,
    ) + bs_ref[...]


def _root_layer(x, wrs, bs):
    n = x.shape[0]
    row_spec = pl.BlockSpec((BLK, H), lambda i: (i, 0))
    return pl.pallas_call(
        _root_body,
        grid=(n // BLK,),
        in_specs=[row_spec, pl.BlockSpec((H, H), lambda i: (0, 0)),
                  pl.BlockSpec((1, H), lambda i: (0, 0))],
        out_specs=row_spec,
        out_shape=jax.ShapeDtypeStruct((n, H), jnp.float32),
    )(x, wrs, bs)


def _dense_body(n_r, relu, *refs):
    idx = 0
    s_refs, c_refs = [], []
    for _ in range(n_r):
        s_refs.append(refs[idx]); idx += 1
        c_refs.append(refs[idx]); idx += 1
    root_ref = refs[idx]; idx += 1
    wl_refs = refs[idx:idx + n_r]; idx += n_r
    out_ref = refs[idx]

    acc = root_ref[...]
    for r in range(n_r):
        cnt = jnp.maximum(c_refs[r][...], 1.0)
        agg = s_refs[r][...] / cnt
        acc = acc + lax.dot_general(
            agg, wl_refs[r][...], (((1,), (0,)), ((), ())),
            preferred_element_type=jnp.float32,
            precision=lax.Precision.HIGHEST,
        )
    if relu:
        acc = jnp.maximum(acc, 0.0)
    out_ref[...] = acc


def _dense_layer(n_r, relu, s_list, cnt_list, root, wl_list):
    n = root.shape[0]
    grid = (n // BLK,)
    row_spec = pl.BlockSpec((BLK, H), lambda i: (i, 0))
    cnt_spec = pl.BlockSpec((BLK, 1), lambda i: (i, 0))
    full_spec = pl.BlockSpec((H, H), lambda i: (0, 0))
    in_specs = []
    args = []
    for r in range(n_r):
        in_specs += [row_spec, cnt_spec]
        args += [s_list[r], cnt_list[r]]
    in_specs += [row_spec] + [full_spec] * n_r
    args += [root] + list(wl_list)
    return pl.pallas_call(
        functools.partial(_dense_body, n_r, relu),
        grid=grid,
        in_specs=in_specs,
        out_specs=row_spec,
        out_shape=jax.ShapeDtypeStruct((n, H), jnp.float32),
    )(*args)


def _layer(xd, eis_padded, consts, params, layer, relu, cnt_prev):
    with_counts = cnt_prev is None
    roots = {}
    for d, rels in DST_GROUPS.items():
        wrs = sum(params[f"Wr{layer}_{r}"] for r in rels)
        bs = sum(params[f"b{layer}_{r}"] for r in rels).reshape(1, H)
        roots[d] = _root_layer(xd[d], wrs, bs)
    s_out, cnt_out = _sc_aggregate(xd, eis_padded, consts, with_counts)
    if cnt_out is None:
        cnt_out = cnt_prev
    out = {}
    for d, rels in DST_GROUPS.items():
        out[d] = _dense_layer(
            len(rels), relu,
            [s_out[r] for r in rels],
            [cnt_out[r][:, :1] for r in rels],
            roots[d],
            [params[f"Wl{layer}_{r}"] for r in rels],
        )
    return out, cnt_out


def kernel(x_show, x_performance, x_song, ei_has_performance, ei_of_song, ei_transitioned_to, ei_setlist_neighbor, ei_rev_has_performance, ei_rev_of_song, ei_rev_transitioned_to, Wl1_has_performance, Wr1_has_performance, b1_has_performance, Wl1_of_song, Wr1_of_song, b1_of_song, Wl1_transitioned_to, Wr1_transitioned_to, b1_transitioned_to, Wl1_setlist_neighbor, Wr1_setlist_neighbor, b1_setlist_neighbor, Wl1_rev_has_performance, Wr1_rev_has_performance, b1_rev_has_performance, Wl1_rev_of_song, Wr1_rev_of_song, b1_rev_of_song, Wl1_rev_transitioned_to, Wr1_rev_transitioned_to, b1_rev_transitioned_to, Wl2_has_performance, Wr2_has_performance, b2_has_performance, Wl2_of_song, Wr2_of_song, b2_of_song, Wl2_transitioned_to, Wr2_transitioned_to, b2_transitioned_to, Wl2_setlist_neighbor, Wr2_setlist_neighbor, b2_setlist_neighbor, Wl2_rev_has_performance, Wr2_rev_has_performance, b2_rev_has_performance, Wl2_rev_of_song, Wr2_rev_of_song, b2_rev_of_song, Wl2_rev_transitioned_to, Wr2_rev_transitioned_to, b2_rev_transitioned_to):
    kw = dict(locals())
    params = {k: v for k, v in kw.items()
              if k[:2] in ("Wl", "Wr") or k[0] == "b"}
    xd = {"show": x_show, "performance": x_performance, "song": x_song}

    pad_n = E_PAD - N_EDGES
    eis_padded = {}
    for (_, r, d) in EDGE_TYPES:
        ei = kw[f"ei_{r}"]
        src_p = jnp.concatenate([ei[0], jnp.zeros((pad_n,), ei.dtype)])
        dst_p = jnp.concatenate([ei[1], jnp.full((pad_n,), -1, ei.dtype)])
        eis_padded[r] = (src_p.astype(jnp.int32).reshape(NW, SLAB, 128),
                         dst_p.astype(jnp.int32).reshape(NW, SLAB, 128))

    consts = (
        jnp.zeros((KB, H), jnp.float32),
        jnp.ones((KB, H), jnp.float32),
    )

    h, cnt = _layer(xd, eis_padded, consts, params, 1, True, None)
    h, _ = _layer(h, eis_padded, consts, params, 2, False, cnt)
    return (h["show"], h["performance"], h["song"])
